# baseline (speedup 1.0000x reference)
import functools

import jax
import jax.numpy as jnp
from jax import lax
from jax.experimental import pallas as pl
from jax.experimental.pallas import tpu as pltpu
from jax.experimental.pallas import tpu_sc as plsc

_NC = 2
_NS = 16
_LANES = 16
_W = 128
_K = 4


def kernel(datasets, perm):
    B, N, D = datasets.shape
    table = datasets.reshape(B * N, D)
    perm_i32 = perm.astype(jnp.int32)
    bpc = B // _NC
    nrounds = bpc // _K
    rows_per_round = _K * N
    stage_rows = rows_per_round // _NS

    mesh = plsc.VectorSubcoreMesh(core_axis_name="c", subcore_axis_name="s")

    @functools.partial(
        pl.kernel,
        out_type=jax.ShapeDtypeStruct((B * N, D), datasets.dtype),
        mesh=mesh,
        scratch_types=[
            pltpu.VMEM((_W,), jnp.int32),
            pltpu.VMEM((_K, _W), jnp.int32),
            pltpu.VMEM((_K, _W, D), jnp.float32),
            pltpu.VMEM_SHARED((rows_per_round, D), jnp.float32),
            pltpu.VMEM_SHARED((rows_per_round, D), jnp.float32),
            pltpu.SemaphoreType.DMA((2,)),
            pltpu.SemaphoreType.DMA((_K,)),
            pltpu.SemaphoreType.DMA((_K,)),
        ],
        compiler_params=pltpu.CompilerParams(use_tc_tiling_on_sc=False),
    )
    def _permute_kernel(table_hbm, perm_hbm, out_hbm,
                        permw_v, idx_v, bufs, slot0, slot1,
                        ssem, gsem, wsem):
        cid = lax.axis_index("c")
        sid = lax.axis_index("s")
        slots = [slot0, slot1]

        pltpu.sync_copy(perm_hbm.at[pl.ds(sid * _W, _W)], permw_v)
        for k in range(_K):
            for kk in range(_W // _LANES):
                sl = pl.ds(kk * _LANES, _LANES)
                idx_v[k, sl] = permw_v[sl] + k * N

        batch0 = cid * bpc

        def stage_src(r):
            row = (batch0 + r * _K) * N + sid * stage_rows
            return table_hbm.at[pl.ds(row, stage_rows)]

        def stage_dst(slot):
            return slot.at[pl.ds(sid * stage_rows, stage_rows)]

        def issue_stage(r, si):
            return pltpu.async_copy(stage_src(r), stage_dst(slots[si]),
                                    ssem.at[si])

        def drain_stage(r, si):
            pltpu.make_async_copy(stage_src(r), stage_dst(slots[si]),
                                  ssem.at[si]).wait()

        def out_dst(r, k):
            row = (batch0 + r * _K + k) * N + sid * _W
            return out_hbm.at[pl.ds(row, _W)]

        def do_round(r, si, stage_next, drain_wb):
            drain_stage(r, si)
            plsc.subcore_barrier()
            if stage_next:
                issue_stage(r + 1, 1 - si)
            gh = [None] * _K
            for k in range(_K):
                if drain_wb:
                    pltpu.make_async_copy(bufs.at[k], out_dst(r, k),
                                          wsem.at[k]).wait()
                gh[k] = pltpu.async_copy(slots[si].at[idx_v.at[k]],
                                         bufs.at[k], gsem.at[k])
            for k in range(_K):
                gh[k].wait()
                pltpu.async_copy(bufs.at[k], out_dst(r, k), wsem.at[k])

        issue_stage(0, 0)
        do_round(0, 0, stage_next=True, drain_wb=False)

        @pl.loop(0, (nrounds - 2) // 2)
        def _rounds(q):
            r = 1 + 2 * q
            do_round(r, 1, stage_next=True, drain_wb=True)
            do_round(r + 1, 0, stage_next=True, drain_wb=True)

        do_round(nrounds - 1, (nrounds - 1) % 2, stage_next=False,
                 drain_wb=True)
        for k in range(_K):
            pltpu.make_async_copy(bufs.at[k], out_dst(nrounds - 1, k),
                                  wsem.at[k]).wait()

    out = _permute_kernel(table, perm_i32)
    return out.reshape(B, N, D)


# speedup vs baseline: 1.0150x; 1.0150x over previous
import functools

import jax
import jax.numpy as jnp
from jax import lax
from jax.experimental import pallas as pl
from jax.experimental.pallas import tpu as pltpu
from jax.experimental.pallas import tpu_sc as plsc

_NC = 2
_NS = 16
_NW = _NC * _NS
_LANES = 16
_W = 128
_NBUF = 8
_LOOKAHEAD = 4
_CHUNK = 32


def kernel(datasets, perm):
    B, N, D = datasets.shape
    table = datasets.reshape(B * N, D)
    cpb = N // _W
    perm2d = perm.astype(jnp.int32).reshape(cpb, _W)
    nb_per_w = B // _NW
    m = nb_per_w * cpb

    mesh = plsc.VectorSubcoreMesh(core_axis_name="c", subcore_axis_name="s")

    @functools.partial(
        pl.kernel,
        out_type=jax.ShapeDtypeStruct((B * N, D), datasets.dtype),
        mesh=mesh,
        scratch_types=[
            pltpu.VMEM((cpb, _W), jnp.int32),
            pltpu.VMEM((m, _W), jnp.int32),
            pltpu.VMEM((_NBUF, _W, D), jnp.float32),
            pltpu.SemaphoreType.DMA((_NBUF,)),
            pltpu.SemaphoreType.DMA((_NBUF,)),
        ],
        compiler_params=pltpu.CompilerParams(use_tc_tiling_on_sc=False),
    )
    def _gather_kernel(table_hbm, perm_hbm, out_hbm,
                       perm_v, idx_v, rows_v, gsem, wsem):
        wid = lax.axis_index("s") * _NC + lax.axis_index("c")
        pltpu.sync_copy(perm_hbm, perm_v)
        b0 = wid * nb_per_w
        row0 = b0 * N

        @pl.loop(0, nb_per_w)
        def _precompute(t):
            base = (b0 + t) * N
            for j in range(cpb):
                for k in range(_W // _LANES):
                    sl = pl.ds(k * _LANES, _LANES)
                    idx_v[t * cpb + j, sl] = perm_v[j, sl] + base

        def g_copy(c, s):
            hs = []
            for g in range(_W // _LANES):
                iv = idx_v[c, pl.ds(g * _LANES, _LANES)]
                hs.append(pltpu.async_copy(
                    table_hbm.at[iv],
                    rows_v.at[s].at[pl.ds(g * _LANES, _LANES)],
                    gsem.at[s]))
            return hs

        def g_wait(hs):
            for h in hs:
                h.wait()

        def w_copy(c, s):
            return pltpu.async_copy(
                rows_v.at[s], out_hbm.at[pl.ds(row0 + c * _W, _W)],
                wsem.at[s])

        @pl.loop(0, m // _CHUNK)
        def _chunk(q):
            c0 = q * _CHUNK
            gh = [None] * _CHUNK
            wh = [None] * _CHUNK
            for s in range(_LOOKAHEAD):
                gh[s] = g_copy(c0 + s, s)
            for p in range(_CHUNK):
                g_wait(gh[p])
                wh[p] = w_copy(c0 + p, p % _NBUF)
                pn = p + _LOOKAHEAD
                if pn < _CHUNK:
                    if p >= _LOOKAHEAD:
                        wh[p - _LOOKAHEAD].wait()
                    gh[pn] = g_copy(c0 + pn, pn % _NBUF)
            for p in range(_CHUNK - _NBUF, _CHUNK):
                wh[p].wait()

    out = _gather_kernel(table, perm2d)
    return out.reshape(B, N, D)
